# baseline (speedup 1.0000x reference)
import functools

import jax
import jax.numpy as jnp
from jax import lax
from jax.experimental import pallas as pl
from jax.experimental.pallas import tpu as pltpu
from jax.experimental.pallas import tpu_sc as plsc

_VOCAB = 100000
_EMBED = 32
_HIDDEN = 128
_CTX = 200

_S_SC = 0
_V_TC = _VOCAB - _S_SC
_VB = 8192
_NB_TC = -(-_V_TC // _VB)
_NB_ALL = -(-_VOCAB // _VB)
_OFF = _S_SC // _VB

_NW = 32
_RPT = _S_SC // _NW
_CH = 256
_NCH = _RPT // _CH


def _gather_h_body(idx_ref, emb_ref, w1t_ref, b1_ref, h_ref, gbuf, sem):
    def issue(t, x):
        r = idx_ref[t]
        rb = pl.multiple_of((r // 8) * 8, 8)
        pltpu.make_async_copy(emb_ref.at[pl.ds(rb, 8), :],
                              gbuf.at[pl.ds(8 * t, 8), :], sem).start()
        return x

    lax.fori_loop(0, _CTX, issue, 0)

    def drain(t, x):
        pltpu.make_async_copy(emb_ref.at[pl.ds(0, 8), :],
                              gbuf.at[pl.ds(0, 8), :], sem).wait()
        return x

    lax.fori_loop(0, _CTX, drain, 0)

    def acc(t, a):
        r = idx_ref[t]
        return a + gbuf[pl.ds(8 * t + (r % 8), 1), :]

    e = lax.fori_loop(0, _CTX, acc, jnp.zeros((1, _EMBED), jnp.float32))
    h = jnp.dot(e, w1t_ref[...], preferred_element_type=jnp.float32) \
        + b1_ref[...]
    h_ref[...] = jnp.maximum(h, 0.0)


def _make_sc_matvec():
    mesh = plsc.VectorSubcoreMesh(core_axis_name="c", subcore_axis_name="s")

    @functools.partial(
        pl.kernel,
        mesh=mesh,
        compiler_params=pltpu.CompilerParams(use_tc_tiling_on_sc=True,
                                             needs_layout_passes=False),
        out_type=(jax.ShapeDtypeStruct((1, _S_SC), jnp.float32),
                  jax.ShapeDtypeStruct((_NW, 16), jnp.float32)),
        scratch_types=[
            pltpu.VMEM((_CH, _HIDDEN), jnp.float32),
            pltpu.VMEM((_CH, _HIDDEN), jnp.float32),
            pltpu.VMEM((_HIDDEN,), jnp.float32),
            pltpu.VMEM((1, _RPT), jnp.float32),
            pltpu.VMEM((_RPT,), jnp.float32),
            pltpu.VMEM((16,), jnp.float32),
            pltpu.SemaphoreType.DMA,
            pltpu.SemaphoreType.DMA,
        ],
    )
    def sc_matvec(h_hbm, w2_hbm, b2_hbm, lg_hbm, st_hbm,
                  buf0, buf1, h_v, lg_v, b2_v, st_v, sem0, sem1):
        wid = lax.axis_index("s") * 2 + lax.axis_index("c")
        r0 = wid * _RPT
        pltpu.sync_copy(h_hbm.at[0], h_v)
        pltpu.sync_copy(b2_hbm.at[pl.ds(r0, _RPT)], b2_v)
        hs = [h_v[pl.ds(16 * c, 16)] for c in range(8)]
        bufs = (buf0, buf1)
        sems = (sem0, sem1)
        lane = lax.iota(jnp.int32, 16)

        pltpu.async_copy(w2_hbm.at[pl.ds(r0, _CH)], buf0, sem0)
        mx = jnp.full((16,), -jnp.inf, jnp.float32)
        for ci in range(_NCH):
            if ci + 1 < _NCH:
                pltpu.async_copy(
                    w2_hbm.at[pl.ds(r0 + (ci + 1) * _CH, _CH)],
                    bufs[(ci + 1) % 2], sems[(ci + 1) % 2])
            buf = bufs[ci % 2]
            pltpu.make_async_copy(
                w2_hbm.at[pl.ds(r0 + ci * _CH, _CH)], buf,
                sems[ci % 2]).wait()

            def group(g, mxc):
                rowvec = jnp.zeros((16,), jnp.float32)
                for j in range(16):
                    r = g * 16 + j
                    acc = buf[r, pl.ds(0, 16)] * hs[0]
                    for c in range(1, 8):
                        acc = acc + buf[r, pl.ds(16 * c, 16)] * hs[c]
                    rowvec = jnp.where(lane == j, jnp.sum(acc), rowvec)
                base = ci * _CH + g * 16
                lg16 = rowvec + b2_v[pl.ds(base, 16)]
                lg_v[0, pl.ds(base, 16)] = lg16
                return jnp.maximum(mxc, lg16)

            mx = lax.fori_loop(0, _CH // 16, group, mx)

        m_t = jnp.max(mx)

        def esum(g, sv):
            return sv + jnp.exp(lg_v[0, pl.ds(16 * g, 16)] - m_t)

        sv = lax.fori_loop(0, _RPT // 16, esum, jnp.zeros((16,), jnp.float32))
        s_t = jnp.sum(sv)

        pltpu.sync_copy(lg_v, lg_hbm.at[pl.ds(0, 1), pl.ds(r0, _RPT)])
        st_v[...] = jnp.where(lane == 0, m_t,
                              jnp.where(lane == 1, s_t, 0.0))
        pltpu.sync_copy(st_v, st_hbm.at[wid])

    return sc_matvec


def _tc_tail_body(h_ref, w2_ref, b2_ref, lg_ref, m_ref, s_ref, m_s, s_s):
    i = pl.program_id(0)

    @pl.when(i == 0)
    def _():
        m_s[0] = -jnp.inf
        s_s[0] = 0.0

    logits = lax.dot_general(
        h_ref[...], w2_ref[...], (((1,), (1,)), ((), ())),
        preferred_element_type=jnp.float32) + b2_ref[...]
    lg_ref[...] = logits

    col = lax.broadcasted_iota(jnp.int32, (1, _VB), 1) + (i + _OFF) * _VB
    lm = jnp.where(col < _VOCAB, logits, -jnp.inf)
    bmax = jnp.max(lm)
    m_old = m_s[0]
    m_new = jnp.maximum(m_old, bmax)
    s_s[0] = s_s[0] * jnp.exp(m_old - m_new) + jnp.sum(jnp.exp(lm - m_new))
    m_s[0] = m_new

    @pl.when(i == pl.num_programs(0) - 1)
    def _():
        m_ref[...] = jnp.full((1, 1), m_s[0], jnp.float32)
        s_ref[...] = jnp.full((1, 1), s_s[0], jnp.float32)


def _normalize_body(tc_ref, m_ref, s_ref, out_ref):
    logz = m_ref[0, 0] + jnp.log(s_ref[0, 0])
    out_ref[...] = tc_ref[...] - logz


def kernel(inputs, emb_table, W1, b1, W2, b2):
    w1t = W1.T
    b1r = b1.reshape(1, _HIDDEN)
    b2r = b2.reshape(1, _VOCAB)

    h = pl.pallas_call(
        _gather_h_body,
        in_specs=[
            pl.BlockSpec(memory_space=pltpu.SMEM),
            pl.BlockSpec(memory_space=pl.ANY),
            pl.BlockSpec((_EMBED, _HIDDEN), lambda: (0, 0)),
            pl.BlockSpec((1, _HIDDEN), lambda: (0, 0)),
        ],
        out_specs=pl.BlockSpec((1, _HIDDEN), lambda: (0, 0)),
        out_shape=jax.ShapeDtypeStruct((1, _HIDDEN), jnp.float32),
        scratch_shapes=[
            pltpu.VMEM((8 * _CTX, _EMBED), jnp.float32),
            pltpu.SemaphoreType.DMA,
        ],
        name="gather_h",
    )(inputs, emb_table, w1t, b1r)

    lg_tc, m_tc, s_tc = pl.pallas_call(
        _tc_tail_body,
        grid=(_NB_TC,),
        in_specs=[
            pl.BlockSpec((1, _HIDDEN), lambda i: (0, 0)),
            pl.BlockSpec((_VB, _HIDDEN), lambda i: (i + _OFF, 0)),
            pl.BlockSpec((1, _VB), lambda i: (0, i + _OFF)),
        ],
        out_specs=[
            pl.BlockSpec((1, _VB), lambda i: (0, i)),
            pl.BlockSpec((1, 1), lambda i: (0, 0)),
            pl.BlockSpec((1, 1), lambda i: (0, 0)),
        ],
        out_shape=[
            jax.ShapeDtypeStruct((1, _V_TC), jnp.float32),
            jax.ShapeDtypeStruct((1, 1), jnp.float32),
            jax.ShapeDtypeStruct((1, 1), jnp.float32),
        ],
        scratch_shapes=[
            pltpu.SMEM((1,), jnp.float32),
            pltpu.SMEM((1,), jnp.float32),
        ],
        name="tc_tail",
    )(h, W2, b2r)

    log_probs = pl.pallas_call(
        _normalize_body,
        grid=(_NB_ALL,),
        in_specs=[
            pl.BlockSpec((1, _VB), lambda j: (0, j)),
            pl.BlockSpec(memory_space=pltpu.SMEM),
            pl.BlockSpec(memory_space=pltpu.SMEM),
        ],
        out_specs=pl.BlockSpec((1, _VB), lambda j: (0, j)),
        out_shape=jax.ShapeDtypeStruct((1, _VOCAB), jnp.float32),
        name="normalize",
    )(lg_tc, m_tc, s_tc)

    return log_probs


# speedup vs baseline: 1.0781x; 1.0781x over previous
import functools

import jax
import jax.numpy as jnp
from jax import lax
from jax.experimental import pallas as pl
from jax.experimental.pallas import tpu as pltpu
from jax.experimental.pallas import tpu_sc as plsc

_VOCAB = 100000
_EMBED = 32
_HIDDEN = 128
_CTX = 200

_S_SC = 0
_V_TC = _VOCAB - _S_SC
_VB = 32768
_NB_TC = -(-_V_TC // _VB)
_NB_ALL = -(-_VOCAB // _VB)
_OFF = _S_SC // _VB

_NW = 32
_RPT = _S_SC // _NW
_CH = 256
_NCH = _RPT // _CH


def _gather_h_body(idx_ref, emb_ref, w1t_ref, b1_ref, h_ref, gbuf, sem):
    def issue(t, x):
        r = idx_ref[t]
        rb = pl.multiple_of((r // 8) * 8, 8)
        pltpu.make_async_copy(emb_ref.at[pl.ds(rb, 8), :],
                              gbuf.at[pl.ds(8 * t, 8), :], sem).start()
        return x

    lax.fori_loop(0, _CTX, issue, 0)

    def drain(t, x):
        pltpu.make_async_copy(emb_ref.at[pl.ds(0, 8), :],
                              gbuf.at[pl.ds(0, 8), :], sem).wait()
        return x

    lax.fori_loop(0, _CTX, drain, 0)

    def acc(t, a):
        r = idx_ref[t]
        return a + gbuf[pl.ds(8 * t + (r % 8), 1), :]

    e = lax.fori_loop(0, _CTX, acc, jnp.zeros((1, _EMBED), jnp.float32))
    h = jnp.dot(e, w1t_ref[...], preferred_element_type=jnp.float32) \
        + b1_ref[...]
    h_ref[...] = jnp.maximum(h, 0.0)


def _make_sc_matvec():
    mesh = plsc.VectorSubcoreMesh(core_axis_name="c", subcore_axis_name="s")

    @functools.partial(
        pl.kernel,
        mesh=mesh,
        compiler_params=pltpu.CompilerParams(use_tc_tiling_on_sc=True,
                                             needs_layout_passes=False),
        out_type=(jax.ShapeDtypeStruct((1, _S_SC), jnp.float32),
                  jax.ShapeDtypeStruct((_NW, 16), jnp.float32)),
        scratch_types=[
            pltpu.VMEM((_CH, _HIDDEN), jnp.float32),
            pltpu.VMEM((_CH, _HIDDEN), jnp.float32),
            pltpu.VMEM((_HIDDEN,), jnp.float32),
            pltpu.VMEM((1, _RPT), jnp.float32),
            pltpu.VMEM((_RPT,), jnp.float32),
            pltpu.VMEM((16,), jnp.float32),
            pltpu.SemaphoreType.DMA,
            pltpu.SemaphoreType.DMA,
        ],
    )
    def sc_matvec(h_hbm, w2_hbm, b2_hbm, lg_hbm, st_hbm,
                  buf0, buf1, h_v, lg_v, b2_v, st_v, sem0, sem1):
        wid = lax.axis_index("s") * 2 + lax.axis_index("c")
        r0 = wid * _RPT
        pltpu.sync_copy(h_hbm.at[0], h_v)
        pltpu.sync_copy(b2_hbm.at[pl.ds(r0, _RPT)], b2_v)
        hs = [h_v[pl.ds(16 * c, 16)] for c in range(8)]
        bufs = (buf0, buf1)
        sems = (sem0, sem1)
        lane = lax.iota(jnp.int32, 16)

        pltpu.async_copy(w2_hbm.at[pl.ds(r0, _CH)], buf0, sem0)
        mx = jnp.full((16,), -jnp.inf, jnp.float32)
        for ci in range(_NCH):
            if ci + 1 < _NCH:
                pltpu.async_copy(
                    w2_hbm.at[pl.ds(r0 + (ci + 1) * _CH, _CH)],
                    bufs[(ci + 1) % 2], sems[(ci + 1) % 2])
            buf = bufs[ci % 2]
            pltpu.make_async_copy(
                w2_hbm.at[pl.ds(r0 + ci * _CH, _CH)], buf,
                sems[ci % 2]).wait()

            def group(g, mxc):
                rowvec = jnp.zeros((16,), jnp.float32)
                for j in range(16):
                    r = g * 16 + j
                    acc = buf[r, pl.ds(0, 16)] * hs[0]
                    for c in range(1, 8):
                        acc = acc + buf[r, pl.ds(16 * c, 16)] * hs[c]
                    rowvec = jnp.where(lane == j, jnp.sum(acc), rowvec)
                base = ci * _CH + g * 16
                lg16 = rowvec + b2_v[pl.ds(base, 16)]
                lg_v[0, pl.ds(base, 16)] = lg16
                return jnp.maximum(mxc, lg16)

            mx = lax.fori_loop(0, _CH // 16, group, mx)

        m_t = jnp.max(mx)

        def esum(g, sv):
            return sv + jnp.exp(lg_v[0, pl.ds(16 * g, 16)] - m_t)

        sv = lax.fori_loop(0, _RPT // 16, esum, jnp.zeros((16,), jnp.float32))
        s_t = jnp.sum(sv)

        pltpu.sync_copy(lg_v, lg_hbm.at[pl.ds(0, 1), pl.ds(r0, _RPT)])
        st_v[...] = jnp.where(lane == 0, m_t,
                              jnp.where(lane == 1, s_t, 0.0))
        pltpu.sync_copy(st_v, st_hbm.at[wid])

    return sc_matvec


def _tc_tail_body(h_ref, w2_ref, b2_ref, lg_ref, m_ref, s_ref, m_s, s_s):
    i = pl.program_id(0)

    @pl.when(i == 0)
    def _():
        m_s[0] = -jnp.inf
        s_s[0] = 0.0

    logits = lax.dot_general(
        h_ref[...], w2_ref[...], (((1,), (1,)), ((), ())),
        preferred_element_type=jnp.float32) + b2_ref[...]
    lg_ref[...] = logits

    col = lax.broadcasted_iota(jnp.int32, (1, _VB), 1) + (i + _OFF) * _VB
    lm = jnp.where(col < _VOCAB, logits, -jnp.inf)
    bmax = jnp.max(lm)
    m_old = m_s[0]
    m_new = jnp.maximum(m_old, bmax)
    s_s[0] = s_s[0] * jnp.exp(m_old - m_new) + jnp.sum(jnp.exp(lm - m_new))
    m_s[0] = m_new

    @pl.when(i == pl.num_programs(0) - 1)
    def _():
        m_ref[...] = jnp.full((1, 1), m_s[0], jnp.float32)
        s_ref[...] = jnp.full((1, 1), s_s[0], jnp.float32)


def _normalize_body(tc_ref, m_ref, s_ref, out_ref):
    logz = m_ref[0, 0] + jnp.log(s_ref[0, 0])
    out_ref[...] = tc_ref[...] - logz


def kernel(inputs, emb_table, W1, b1, W2, b2):
    w1t = W1.T
    b1r = b1.reshape(1, _HIDDEN)
    b2r = b2.reshape(1, _VOCAB)

    h = pl.pallas_call(
        _gather_h_body,
        in_specs=[
            pl.BlockSpec(memory_space=pltpu.SMEM),
            pl.BlockSpec(memory_space=pl.ANY),
            pl.BlockSpec((_EMBED, _HIDDEN), lambda: (0, 0)),
            pl.BlockSpec((1, _HIDDEN), lambda: (0, 0)),
        ],
        out_specs=pl.BlockSpec((1, _HIDDEN), lambda: (0, 0)),
        out_shape=jax.ShapeDtypeStruct((1, _HIDDEN), jnp.float32),
        scratch_shapes=[
            pltpu.VMEM((8 * _CTX, _EMBED), jnp.float32),
            pltpu.SemaphoreType.DMA,
        ],
        name="gather_h",
    )(inputs, emb_table, w1t, b1r)

    lg_tc, m_tc, s_tc = pl.pallas_call(
        _tc_tail_body,
        grid=(_NB_TC,),
        in_specs=[
            pl.BlockSpec((1, _HIDDEN), lambda i: (0, 0)),
            pl.BlockSpec((_VB, _HIDDEN), lambda i: (i + _OFF, 0)),
            pl.BlockSpec((1, _VB), lambda i: (0, i + _OFF)),
        ],
        out_specs=[
            pl.BlockSpec((1, _VB), lambda i: (0, i)),
            pl.BlockSpec((1, 1), lambda i: (0, 0)),
            pl.BlockSpec((1, 1), lambda i: (0, 0)),
        ],
        out_shape=[
            jax.ShapeDtypeStruct((1, _V_TC), jnp.float32),
            jax.ShapeDtypeStruct((1, 1), jnp.float32),
            jax.ShapeDtypeStruct((1, 1), jnp.float32),
        ],
        scratch_shapes=[
            pltpu.SMEM((1,), jnp.float32),
            pltpu.SMEM((1,), jnp.float32),
        ],
        name="tc_tail",
    )(h, W2, b2r)

    log_probs = pl.pallas_call(
        _normalize_body,
        grid=(_NB_ALL,),
        in_specs=[
            pl.BlockSpec((1, _VB), lambda j: (0, j)),
            pl.BlockSpec(memory_space=pltpu.SMEM),
            pl.BlockSpec(memory_space=pltpu.SMEM),
        ],
        out_specs=pl.BlockSpec((1, _VB), lambda j: (0, j)),
        out_shape=jax.ShapeDtypeStruct((1, _VOCAB), jnp.float32),
        name="normalize",
    )(lg_tc, m_tc, s_tc)

    return log_probs


# speedup vs baseline: 1.2062x; 1.1188x over previous
import functools

import jax
import jax.numpy as jnp
from jax import lax
from jax.experimental import pallas as pl
from jax.experimental.pallas import tpu as pltpu
from jax.experimental.pallas import tpu_sc as plsc

_VOCAB = 100000
_EMBED = 32
_HIDDEN = 128
_CTX = 200

_S_SC = 0
_V_TC = _VOCAB - _S_SC
_VB = 16384
_NB_TC = -(-_V_TC // _VB)
_NB_ALL = -(-_VOCAB // _VB)
_OFF = _S_SC // _VB

_NW = 32
_RPT = _S_SC // _NW
_CH = 256
_NCH = _RPT // _CH


def _gather_h_body(idx_ref, emb_ref, w1t_ref, b1_ref, h_ref, gbuf, sem):
    def issue(t, x):
        r = idx_ref[t]
        rb = pl.multiple_of((r // 8) * 8, 8)
        pltpu.make_async_copy(emb_ref.at[pl.ds(rb, 8), :],
                              gbuf.at[pl.ds(8 * t, 8), :], sem).start()
        return x

    lax.fori_loop(0, _CTX, issue, 0)

    def drain(t, x):
        pltpu.make_async_copy(emb_ref.at[pl.ds(0, 8), :],
                              gbuf.at[pl.ds(0, 8), :], sem).wait()
        return x

    lax.fori_loop(0, _CTX, drain, 0)

    def acc(t, a):
        r = idx_ref[t]
        return a + gbuf[pl.ds(8 * t + (r % 8), 1), :]

    e = lax.fori_loop(0, _CTX, acc, jnp.zeros((1, _EMBED), jnp.float32))
    h = jnp.dot(e, w1t_ref[...], preferred_element_type=jnp.float32) \
        + b1_ref[...]
    h_ref[...] = jnp.maximum(h, 0.0)


def _make_sc_matvec():
    mesh = plsc.VectorSubcoreMesh(core_axis_name="c", subcore_axis_name="s")

    @functools.partial(
        pl.kernel,
        mesh=mesh,
        compiler_params=pltpu.CompilerParams(use_tc_tiling_on_sc=True,
                                             needs_layout_passes=False),
        out_type=(jax.ShapeDtypeStruct((1, _S_SC), jnp.float32),
                  jax.ShapeDtypeStruct((_NW, 16), jnp.float32)),
        scratch_types=[
            pltpu.VMEM((_CH, _HIDDEN), jnp.float32),
            pltpu.VMEM((_CH, _HIDDEN), jnp.float32),
            pltpu.VMEM((_HIDDEN,), jnp.float32),
            pltpu.VMEM((1, _RPT), jnp.float32),
            pltpu.VMEM((_RPT,), jnp.float32),
            pltpu.VMEM((16,), jnp.float32),
            pltpu.SemaphoreType.DMA,
            pltpu.SemaphoreType.DMA,
        ],
    )
    def sc_matvec(h_hbm, w2_hbm, b2_hbm, lg_hbm, st_hbm,
                  buf0, buf1, h_v, lg_v, b2_v, st_v, sem0, sem1):
        wid = lax.axis_index("s") * 2 + lax.axis_index("c")
        r0 = wid * _RPT
        pltpu.sync_copy(h_hbm.at[0], h_v)
        pltpu.sync_copy(b2_hbm.at[pl.ds(r0, _RPT)], b2_v)
        hs = [h_v[pl.ds(16 * c, 16)] for c in range(8)]
        bufs = (buf0, buf1)
        sems = (sem0, sem1)
        lane = lax.iota(jnp.int32, 16)

        pltpu.async_copy(w2_hbm.at[pl.ds(r0, _CH)], buf0, sem0)
        mx = jnp.full((16,), -jnp.inf, jnp.float32)
        for ci in range(_NCH):
            if ci + 1 < _NCH:
                pltpu.async_copy(
                    w2_hbm.at[pl.ds(r0 + (ci + 1) * _CH, _CH)],
                    bufs[(ci + 1) % 2], sems[(ci + 1) % 2])
            buf = bufs[ci % 2]
            pltpu.make_async_copy(
                w2_hbm.at[pl.ds(r0 + ci * _CH, _CH)], buf,
                sems[ci % 2]).wait()

            def group(g, mxc):
                rowvec = jnp.zeros((16,), jnp.float32)
                for j in range(16):
                    r = g * 16 + j
                    acc = buf[r, pl.ds(0, 16)] * hs[0]
                    for c in range(1, 8):
                        acc = acc + buf[r, pl.ds(16 * c, 16)] * hs[c]
                    rowvec = jnp.where(lane == j, jnp.sum(acc), rowvec)
                base = ci * _CH + g * 16
                lg16 = rowvec + b2_v[pl.ds(base, 16)]
                lg_v[0, pl.ds(base, 16)] = lg16
                return jnp.maximum(mxc, lg16)

            mx = lax.fori_loop(0, _CH // 16, group, mx)

        m_t = jnp.max(mx)

        def esum(g, sv):
            return sv + jnp.exp(lg_v[0, pl.ds(16 * g, 16)] - m_t)

        sv = lax.fori_loop(0, _RPT // 16, esum, jnp.zeros((16,), jnp.float32))
        s_t = jnp.sum(sv)

        pltpu.sync_copy(lg_v, lg_hbm.at[pl.ds(0, 1), pl.ds(r0, _RPT)])
        st_v[...] = jnp.where(lane == 0, m_t,
                              jnp.where(lane == 1, s_t, 0.0))
        pltpu.sync_copy(st_v, st_hbm.at[wid])

    return sc_matvec


def _matvec_body(h_ref, w2_ref, b2_ref, out_ref, lgs, m_s, s_s):
    i = pl.program_id(0)

    @pl.when(i == 0)
    def _():
        m_s[0] = -jnp.inf
        s_s[0] = 0.0

    @pl.when(i < _NB_ALL)
    def _():
        logits = lax.dot_general(
            h_ref[...], w2_ref[...], (((1,), (1,)), ((), ())),
            preferred_element_type=jnp.float32) \
            + b2_ref[...].reshape(1, _VB)
        lgs[i] = logits

        col = lax.broadcasted_iota(jnp.int32, (1, _VB), 1) + i * _VB
        lm = jnp.where(col < _VOCAB, logits, -jnp.inf)
        bmax = jnp.max(lm)
        m_old = m_s[0]
        m_new = jnp.maximum(m_old, bmax)
        s_s[0] = s_s[0] * jnp.exp(m_old - m_new) \
            + jnp.sum(jnp.exp(lm - m_new))
        m_s[0] = m_new

    @pl.when(i >= _NB_ALL)
    def _():
        logz = m_s[0] + jnp.log(s_s[0])
        out_ref[...] = lgs[i - _NB_ALL] - logz


def kernel(inputs, emb_table, W1, b1, W2, b2):
    w1t = W1.T
    b1r = b1.reshape(1, _HIDDEN)

    h = pl.pallas_call(
        _gather_h_body,
        in_specs=[
            pl.BlockSpec(memory_space=pltpu.SMEM),
            pl.BlockSpec(memory_space=pl.ANY),
            pl.BlockSpec((_EMBED, _HIDDEN), lambda: (0, 0)),
            pl.BlockSpec((1, _HIDDEN), lambda: (0, 0)),
        ],
        out_specs=pl.BlockSpec((1, _HIDDEN), lambda: (0, 0)),
        out_shape=jax.ShapeDtypeStruct((1, _HIDDEN), jnp.float32),
        scratch_shapes=[
            pltpu.VMEM((8 * _CTX, _EMBED), jnp.float32),
            pltpu.SemaphoreType.DMA,
        ],
        name="gather_h",
    )(inputs, emb_table, w1t, b1r)

    log_probs = pl.pallas_call(
        _matvec_body,
        grid=(2 * _NB_ALL,),
        in_specs=[
            pl.BlockSpec((1, _HIDDEN), lambda i: (0, 0)),
            pl.BlockSpec((_VB, _HIDDEN),
                         lambda i: (jnp.minimum(i, _NB_ALL - 1), 0)),
            pl.BlockSpec((_VB,), lambda i: (jnp.minimum(i, _NB_ALL - 1),)),
        ],
        out_specs=pl.BlockSpec((1, _VB),
                               lambda i: (0, jnp.maximum(i - _NB_ALL, 0))),
        out_shape=jax.ShapeDtypeStruct((1, _VOCAB), jnp.float32),
        scratch_shapes=[
            pltpu.VMEM((_NB_ALL, 1, _VB), jnp.float32),
            pltpu.SMEM((1,), jnp.float32),
            pltpu.SMEM((1,), jnp.float32),
        ],
        name="matvec_lsm",
    )(h, W2, b2)

    return log_probs
